# baseline (device time: 114987 ns/iter reference)
import jax
import jax.numpy as jnp
from jax import lax
from jax.experimental import pallas as pl
from jax.experimental.pallas import tpu as pltpu

M = 2048
K = 8192
N = 2048
MZ = M // 2
NC = 8
CN = N // NC
WR = 128
NW = N // WR
DR = 128
ND = MZ // DR

BF = jnp.bfloat16
F32 = jnp.float32


def kernel(dy, W):
    def body(dy_hbm, w_hbm, out_hbm,
             stage, dy_bf, w_bf, p_buf, ysend, yrecv, zsend, zrecv, zf32,
             stage_sems, ysend_sems, yrecv_sems, zsend_sems, zrecv_sems,
             oy_sems, oz_sems):
        my_x = lax.axis_index("x")
        my_y = lax.axis_index("y")
        my_z = lax.axis_index("z")
        y_nbr = (my_x, 1 - my_y, my_z)
        z_nbr = (my_x, my_y, 1 - my_z)
        row0 = my_z * MZ

        def rem2(i):
            return lax.rem(i, 2) if not isinstance(i, int) else i % 2

        def rem4(i):
            return lax.rem(i, 4) if not isinstance(i, int) else i % 4

        def dy_dma(i):
            return pltpu.make_async_copy(
                dy_hbm.at[pl.ds(row0 + i * DR, DR), :],
                stage.at[pl.ds((i % 2) * WR, DR), :],
                stage_sems.at[i % 2],
            )

        def w_dma(j, s):
            return pltpu.make_async_copy(
                w_hbm.at[pl.ds(j * WR, WR), :],
                stage.at[pl.ds(s * WR, WR), :],
                stage_sems.at[s],
            )

        def oy_dma(k):
            dk = rem2(k // 2 if isinstance(k, int) else lax.div(k, 2))
            return pltpu.make_async_copy(
                p_buf.at[pl.ds(dk * MZ, MZ), pl.ds(rem2(k) * CN, CN)],
                out_hbm.at[pl.ds(row0, MZ), pl.ds(k * CN, CN)],
                oy_sems.at[rem2(k)],
            )

        def oz_dma(k):
            return pltpu.make_async_copy(
                zf32.at[pl.ds(rem2(k) * MZ, MZ), :],
                out_hbm.at[pl.ds((1 - my_z) * MZ, MZ), pl.ds(k * CN, CN)],
                oz_sems.at[rem2(k)],
            )

        def y_rdma(c):
            return pltpu.make_async_remote_copy(
                src_ref=ysend.at[pl.ds(rem4(c) * MZ, MZ), :],
                dst_ref=yrecv.at[pl.ds(c * MZ, MZ), :],
                send_sem=ysend_sems.at[rem4(c)],
                recv_sem=yrecv_sems.at[c],
                device_id=y_nbr,
                device_id_type=pl.DeviceIdType.MESH,
            )

        def z_rdma(c):
            return pltpu.make_async_remote_copy(
                src_ref=zsend.at[pl.ds(rem2(c) * MZ, MZ), :],
                dst_ref=zrecv.at[pl.ds(c * MZ, MZ), :],
                send_sem=zsend_sems.at[rem2(c)],
                recv_sem=zrecv_sems.at[c],
                device_id=z_nbr,
                device_id_type=pl.DeviceIdType.MESH,
            )

        dy_dma(0).start()

        barrier = pltpu.get_barrier_semaphore()
        for nbr in (y_nbr, z_nbr):
            pl.semaphore_signal(
                barrier, inc=1, device_id=nbr,
                device_id_type=pl.DeviceIdType.MESH,
            )
        pl.semaphore_wait(barrier, 2)

        for i in range(ND):
            if i + 1 < ND:
                dy_dma(i + 1).start()
            dy_dma(i).wait()
            dy_bf[pl.ds(i * DR, DR), :] = stage[pl.ds((i % 2) * WR, DR), :].astype(BF)
            if i == ND - 2:
                w_dma(0, 0).start()
            if i == ND - 1:
                w_dma(1, 1).start()

        def loop_body(c, carry):
            odd = lax.rem(c, 2) == 1

            @pl.when(c < NC)
            def _wstream():
                for jj in (0, 1):
                    j = 2 * c + jj
                    w_dma(j, jj).wait()
                    w_bf[pl.ds(lax.rem(j, 8) * WR, WR), :] = (
                        stage[pl.ds(jj * WR, WR), :].astype(BF)
                    )

                    @pl.when(c + 1 < NC)
                    def _():
                        w_dma(j + 2, jj).start()

            @pl.when((c < NC) & odd)
            def _compute():
                @pl.when(c >= 5)
                def _():
                    oy_dma(c - 5).wait()
                    oy_dma(c - 4).wait()
                    y_rdma(c - 5).wait_send()
                    y_rdma(c - 4).wait_send()

                dslot = lax.rem(lax.div(c, 2), 2)
                woff = lax.rem(c - 1, 4) * CN
                p = lax.dot_general(
                    dy_bf[...], w_bf[pl.ds(woff, 2 * CN), :],
                    dimension_numbers=(((1,), (1,)), ((), ())),
                    preferred_element_type=F32,
                )
                p_buf[pl.ds(dslot * MZ, MZ), :] = p
                ysend[pl.ds(lax.rem(c - 1, 4) * MZ, MZ), :] = p[:, :CN].astype(BF)
                ysend[pl.ds(lax.rem(c, 4) * MZ, MZ), :] = p[:, CN:].astype(BF)
                y_rdma(c - 1).start()
                y_rdma(c).start()

            @pl.when((c >= 2) & (c <= NC + 1))
            def _reduce():
                k = c - 2
                dk = lax.rem(lax.div(k, 2), 2)
                col = lax.rem(k, 2) * CN
                y_rdma(k).wait_recv()
                pk = (
                    p_buf[pl.ds(dk * MZ, MZ), pl.ds(col, CN)]
                    + yrecv[pl.ds(k * MZ, MZ), :].astype(F32)
                )
                p_buf[pl.ds(dk * MZ, MZ), pl.ds(col, CN)] = pk

                @pl.when(k >= 2)
                def _():
                    z_rdma(k - 2).wait_send()

                zsend[pl.ds(lax.rem(k, 2) * MZ, MZ), :] = pk.astype(BF)
                z_rdma(k).start()
                oy_dma(k).start()

            @pl.when(c >= 3)
            def _gather():
                j = c - 3
                z_rdma(j).wait_recv()

                @pl.when(j >= 2)
                def _():
                    oz_dma(j - 2).wait()

                zf32[pl.ds(lax.rem(j, 2) * MZ, MZ), :] = (
                    zrecv[pl.ds(j * MZ, MZ), :].astype(F32)
                )
                oz_dma(j).start()

            return carry

        lax.fori_loop(0, NC + 3, loop_body, 0)

        for k in (NC - 4, NC - 3, NC - 2, NC - 1):
            y_rdma(k).wait_send()
            oy_dma(k).wait()
        for k in (NC - 2, NC - 1):
            z_rdma(k).wait_send()
            oz_dma(k).wait()

    return pl.pallas_call(
        body,
        out_shape=jax.ShapeDtypeStruct((M, N), F32),
        in_specs=[
            pl.BlockSpec(memory_space=pl.ANY),
            pl.BlockSpec(memory_space=pl.ANY),
        ],
        out_specs=pl.BlockSpec(memory_space=pl.ANY),
        scratch_shapes=[
            pltpu.VMEM((2 * WR, K), F32),
            pltpu.VMEM((MZ, K), BF),
            pltpu.VMEM((4 * CN, K), BF),
            pltpu.VMEM((2 * MZ, 2 * CN), F32),
            pltpu.VMEM((4 * MZ, CN), BF),
            pltpu.VMEM((NC * MZ, CN), BF),
            pltpu.VMEM((2 * MZ, CN), BF),
            pltpu.VMEM((NC * MZ, CN), BF),
            pltpu.VMEM((2 * MZ, CN), F32),
            pltpu.SemaphoreType.DMA((2,)),
            pltpu.SemaphoreType.DMA((4,)),
            pltpu.SemaphoreType.DMA((NC,)),
            pltpu.SemaphoreType.DMA((2,)),
            pltpu.SemaphoreType.DMA((NC,)),
            pltpu.SemaphoreType.DMA((2,)),
            pltpu.SemaphoreType.DMA((2,)),
        ],
        compiler_params=pltpu.CompilerParams(
            collective_id=0,
            vmem_limit_bytes=63 * 1024 * 1024,
        ),
    )(dy, W)
